# baseline (speedup 1.0000x reference)
import functools

import jax
import jax.numpy as jnp
from jax import lax
from jax.experimental import pallas as pl
from jax.experimental.pallas import tpu as pltpu
from jax.experimental.pallas import tpu_sc as plsc

NUM_CAMERAS = 700000
CAMERA_COUNT = 6
SEQ_LEN = 100000
BOUNDARY = SEQ_LEN * CAMERA_COUNT
N_IDX = 1048576

NC, NS, L = 2, 16, 16
NW = NC * NS

T_ROWS = 100352
T_PER_W = T_ROWS // NW
T_GROUPS = T_PER_W // L

IDX_PER_W = N_IDX // NW
CHUNK = 2048
N_CHUNKS = IDX_PER_W // CHUNK
GATHER = 128
N_GATHER = CHUNK // GATHER


def _expmap_table_body(tang_hbm, table_hbm, in_v, out_v, sem):
    wid = lax.axis_index("s") * NC + lax.axis_index("c")
    base = wid * T_PER_W
    pltpu.sync_copy(tang_hbm.at[pl.ds(base * 6, T_PER_W * 6)], in_v)

    iota = lax.iota(jnp.int32, L)

    def group(g, _):
        r = iota + g * L
        r6 = r * 6
        c = [plsc.load_gather(in_v, [r6 + j]) for j in range(6)]
        tx, ty, tz, wx, wy, wz = c
        n = wx * wx + wy * wy + wz * wz
        u = jnp.maximum(n, 1e-4)
        f1 = 1.0 + u * (-1.0 / 6 + u * (1.0 / 120 + u * (-1.0 / 5040 + u * (1.0 / 362880))))
        f2 = 0.5 + u * (-1.0 / 24 + u * (1.0 / 720 + u * (-1.0 / 40320 + u * (1.0 / 3628800))))
        xy = f2 * wx * wy
        xz = f2 * wx * wz
        yz = f2 * wy * wz
        ax, ay, az = f1 * wx, f1 * wy, f1 * wz
        vals = (
            1.0 + f2 * (wx * wx - n), xy - az, xz + ay, tx,
            xy + az, 1.0 + f2 * (wy * wy - n), yz - ax, ty,
            xz - ay, yz + ax, 1.0 + f2 * (wz * wz - n), tz,
        )
        r16 = r * 16
        for j, val in enumerate(vals):
            plsc.store_scatter(out_v, [r16 + j], val)
        return 0

    lax.fori_loop(0, T_GROUPS, group, 0)
    pltpu.sync_copy(out_v, table_hbm.at[pl.ds(base * 16, T_PER_W * 16)])


def _gather_body(idx_hbm, table_hbm, out_hbm, idxraw_v, midx_v, rows_v, sem):
    cid = lax.axis_index("c")
    sid = lax.axis_index("s")
    wid = sid * NC + cid
    def chunk_body(ci, _):
        base = wid * IDX_PER_W + ci * CHUNK
        pltpu.sync_copy(idx_hbm.at[pl.ds(base, CHUNK)], idxraw_v)

        def map_group(g, _):
            v = idxraw_v[pl.ds(g * L, L)]
            cam = (v >= SEQ_LEN).astype(jnp.int32)
            for k in range(2, CAMERA_COUNT):
                cam = cam + (v >= k * SEQ_LEN).astype(jnp.int32)
            row = jnp.where(v < BOUNDARY, cam, v - (BOUNDARY - CAMERA_COUNT))
            midx_v[pl.ds(g * L, L)] = row
            return 0

        lax.fori_loop(0, CHUNK // L, map_group, 0)

        cps = [pltpu.async_copy(
            table_hbm.at[midx_v.at[pl.ds(j * GATHER, GATHER)]],
            rows_v.at[pl.ds(j * GATHER, GATHER)], sem)
            for j in range(N_GATHER)]
        for cp in cps:
            cp.wait()
        pltpu.sync_copy(rows_v, out_hbm.at[pl.ds(base, CHUNK)])
        return 0

    lax.fori_loop(0, N_CHUNKS, chunk_body, 0)


_MESH = plsc.VectorSubcoreMesh(core_axis_name="c", subcore_axis_name="s")
_PARAMS = pltpu.CompilerParams(needs_layout_passes=False,
                               use_tc_tiling_on_sc=False)

_expmap_table = functools.partial(
    pl.kernel,
    out_type=jax.ShapeDtypeStruct((T_ROWS * 16,), jnp.float32),
    mesh=_MESH,
    compiler_params=_PARAMS,
    scratch_types=[
        pltpu.VMEM((T_PER_W * 6,), jnp.float32),
        pltpu.VMEM((T_PER_W * 16,), jnp.float32),
        pltpu.SemaphoreType.DMA,
    ],
)(_expmap_table_body)

_gather = functools.partial(
    pl.kernel,
    out_type=jax.ShapeDtypeStruct((N_IDX, 16), jnp.float32),
    mesh=_MESH,
    compiler_params=_PARAMS,
    scratch_types=[
        pltpu.VMEM((CHUNK,), jnp.int32),
        pltpu.VMEM((CHUNK,), jnp.int32),
        pltpu.VMEM((CHUNK, 16), jnp.float32),
        pltpu.SemaphoreType.DMA,
    ],
)(_gather_body)


def kernel(indices, ext_adjustment, lidar_adjustment):
    tang = jnp.concatenate(
        [ext_adjustment, lidar_adjustment,
         jnp.zeros((T_ROWS - CAMERA_COUNT - SEQ_LEN, 6), ext_adjustment.dtype)],
        axis=0).reshape(T_ROWS * 6)
    table = _expmap_table(tang).reshape(T_ROWS, 16)
    out = _gather(indices.astype(jnp.int32), table)
    return out[:, :12].reshape(N_IDX, 3, 4)


# speedup vs baseline: 6.6423x; 6.6423x over previous
import functools

import jax
import jax.numpy as jnp
from jax import lax
from jax.experimental import pallas as pl
from jax.experimental.pallas import tpu as pltpu
from jax.experimental.pallas import tpu_sc as plsc

NUM_CAMERAS = 700000
CAMERA_COUNT = 6
SEQ_LEN = 100000
BOUNDARY = SEQ_LEN * CAMERA_COUNT
N_IDX = 1048576

NC, NS, L = 2, 16, 16
NW = NC * NS

REP = 2048
LOG_REP = 11
EXT_ROWS = CAMERA_COUNT * REP
T_ROWS = 112640
T_PER_W = T_ROWS // NW
T_GROUPS = T_PER_W // L
T_SRC = 100368
LID_ROWS = 3528
IN_ROWS = 8 + LID_ROWS

IDX_PER_W = N_IDX // NW
CHUNK = 2048
N_CHUNKS = IDX_PER_W // CHUNK
GATHER = 128
N_GATHER = CHUNK // GATHER


def _expmap_table_body(tang_hbm, table_hbm, in_v, out_v, sem):
    wid = lax.axis_index("s") * NC + lax.axis_index("c")
    base = wid * T_PER_W
    o_w = jnp.maximum(0, base - (EXT_ROWS - CAMERA_COUNT))
    o_al = (o_w >> 2) << 2
    o6 = pl.multiple_of(o_al * 6, 8)
    pltpu.sync_copy(tang_hbm.at[pl.ds(0, 48)], in_v.at[pl.ds(0, 48)])
    pltpu.sync_copy(tang_hbm.at[pl.ds(o6, LID_ROWS * 6)],
                    in_v.at[pl.ds(48, LID_ROWS * 6)])

    iota = lax.iota(jnp.int32, L)

    def group(g, _):
        rloc = iota + g * L
        rglob = rloc + base
        loc = jnp.where(rglob < EXT_ROWS, rglob >> LOG_REP,
                        rglob - (EXT_ROWS - CAMERA_COUNT) - o_al + 8)
        l6 = loc * 6
        c = [plsc.load_gather(in_v, [l6 + j]) for j in range(6)]
        tx, ty, tz, wx, wy, wz = c
        n = wx * wx + wy * wy + wz * wz
        u = jnp.maximum(n, 1e-4)
        f1 = 1.0 + u * (-1.0 / 6 + u * (1.0 / 120 + u * (-1.0 / 5040 + u * (1.0 / 362880))))
        f2 = 0.5 + u * (-1.0 / 24 + u * (1.0 / 720 + u * (-1.0 / 40320 + u * (1.0 / 3628800))))
        xy = f2 * wx * wy
        xz = f2 * wx * wz
        yz = f2 * wy * wz
        ax, ay, az = f1 * wx, f1 * wy, f1 * wz
        vals = (
            1.0 + f2 * (wx * wx - n), xy - az, xz + ay, tx,
            xy + az, 1.0 + f2 * (wy * wy - n), yz - ax, ty,
            xz - ay, yz + ax, 1.0 + f2 * (wz * wz - n), tz,
        )
        r16 = rloc * 16
        for j, val in enumerate(vals):
            plsc.store_scatter(out_v, [r16 + j], val)
        return 0

    lax.fori_loop(0, T_GROUPS, group, 0)
    pltpu.sync_copy(out_v, table_hbm.at[pl.ds(base * 16, T_PER_W * 16)])


def _gather_body(idx_hbm, table_hbm, out_hbm, idxraw_v, midx_v, rows_v, sem):
    cid = lax.axis_index("c")
    sid = lax.axis_index("s")
    wid = sid * NC + cid

    def chunk_body(ci, _):
        base = wid * IDX_PER_W + ci * CHUNK
        pltpu.sync_copy(idx_hbm.at[pl.ds(base, CHUNK)], idxraw_v)

        def map_group(g, _):
            v = idxraw_v[pl.ds(g * L, L)]
            cam = (v >= SEQ_LEN).astype(jnp.int32)
            for k in range(2, CAMERA_COUNT):
                cam = cam + (v >= k * SEQ_LEN).astype(jnp.int32)
            ext_row = (cam << LOG_REP) | (v & (REP - 1))
            row = jnp.where(v < BOUNDARY, ext_row, v - (BOUNDARY - EXT_ROWS))
            midx_v[pl.ds(g * L, L)] = row
            return 0

        lax.fori_loop(0, CHUNK // L, map_group, 0)

        cps = [pltpu.async_copy(
            table_hbm.at[midx_v.at[pl.ds(j * GATHER, GATHER)]],
            rows_v.at[pl.ds(j * GATHER, GATHER)], sem)
            for j in range(N_GATHER)]
        for cp in cps:
            cp.wait()
        pltpu.sync_copy(rows_v, out_hbm.at[pl.ds(base, CHUNK)])
        return 0

    lax.fori_loop(0, N_CHUNKS, chunk_body, 0)


_MESH = plsc.VectorSubcoreMesh(core_axis_name="c", subcore_axis_name="s")
_PARAMS = pltpu.CompilerParams(needs_layout_passes=False,
                               use_tc_tiling_on_sc=False)

_expmap_table = functools.partial(
    pl.kernel,
    out_type=jax.ShapeDtypeStruct((T_ROWS * 16,), jnp.float32),
    mesh=_MESH,
    compiler_params=_PARAMS,
    scratch_types=[
        pltpu.VMEM((IN_ROWS * 6,), jnp.float32),
        pltpu.VMEM((T_PER_W * 16,), jnp.float32),
        pltpu.SemaphoreType.DMA,
    ],
)(_expmap_table_body)

_gather = functools.partial(
    pl.kernel,
    out_type=jax.ShapeDtypeStruct((N_IDX, 16), jnp.float32),
    mesh=_MESH,
    compiler_params=_PARAMS,
    scratch_types=[
        pltpu.VMEM((CHUNK,), jnp.int32),
        pltpu.VMEM((CHUNK,), jnp.int32),
        pltpu.VMEM((CHUNK, 16), jnp.float32),
        pltpu.SemaphoreType.DMA,
    ],
)(_gather_body)


def kernel(indices, ext_adjustment, lidar_adjustment):
    tang = jnp.concatenate(
        [ext_adjustment, lidar_adjustment,
         jnp.zeros((T_SRC - CAMERA_COUNT - SEQ_LEN, 6), ext_adjustment.dtype)],
        axis=0).reshape(T_SRC * 6)
    table = _expmap_table(tang).reshape(T_ROWS, 16)
    out = _gather(indices.astype(jnp.int32), table)
    return out[:, :12].reshape(N_IDX, 3, 4)


# speedup vs baseline: 6.6605x; 1.0027x over previous
import functools

import jax
import jax.numpy as jnp
from jax import lax
from jax.experimental import pallas as pl
from jax.experimental.pallas import tpu as pltpu
from jax.experimental.pallas import tpu_sc as plsc

NUM_CAMERAS = 700000
CAMERA_COUNT = 6
SEQ_LEN = 100000
BOUNDARY = SEQ_LEN * CAMERA_COUNT
N_IDX = 1048576

NC, NS, L = 2, 16, 16
NW = NC * NS

EXT_SHIFT = 5
EXT_DIV = SEQ_LEN >> EXT_SHIFT
EXT_ROWS = 18752
T_ROWS = 118784
T_PER_W = T_ROWS // NW
T_GROUPS = T_PER_W // L
T_SRC = 100368
LID_ROWS = 3720
IN_ROWS = 8 + LID_ROWS

IDX_PER_W = N_IDX // NW
CHUNK = 2048
N_CHUNKS = IDX_PER_W // CHUNK
GATHER = 128
N_GATHER = CHUNK // GATHER


def _expmap_table_body(tang_hbm, table_hbm, in_v, out_v, sem):
    wid = lax.axis_index("s") * NC + lax.axis_index("c")
    base = wid * T_PER_W
    o_w = jnp.maximum(0, base - (EXT_ROWS - CAMERA_COUNT))
    o_al = (o_w >> 2) << 2
    o6 = pl.multiple_of(o_al * 6, 8)
    pltpu.sync_copy(tang_hbm.at[pl.ds(0, 48)], in_v.at[pl.ds(0, 48)])
    pltpu.sync_copy(tang_hbm.at[pl.ds(o6, LID_ROWS * 6)],
                    in_v.at[pl.ds(48, LID_ROWS * 6)])

    iota = lax.iota(jnp.int32, L)

    def group(g, _):
        rloc = iota + g * L
        rglob = rloc + base
        loc = jnp.where(rglob < EXT_ROWS, rglob // EXT_DIV,
                        rglob - (EXT_ROWS - CAMERA_COUNT) - o_al + 8)
        l6 = loc * 6
        c = [plsc.load_gather(in_v, [l6 + j]) for j in range(6)]
        tx, ty, tz, wx, wy, wz = c
        n = wx * wx + wy * wy + wz * wz
        u = jnp.maximum(n, 1e-4)
        f1 = 1.0 + u * (-1.0 / 6 + u * (1.0 / 120 + u * (-1.0 / 5040 + u * (1.0 / 362880))))
        f2 = 0.5 + u * (-1.0 / 24 + u * (1.0 / 720 + u * (-1.0 / 40320 + u * (1.0 / 3628800))))
        xy = f2 * wx * wy
        xz = f2 * wx * wz
        yz = f2 * wy * wz
        ax, ay, az = f1 * wx, f1 * wy, f1 * wz
        vals = (
            1.0 + f2 * (wx * wx - n), xy - az, xz + ay, tx,
            xy + az, 1.0 + f2 * (wy * wy - n), yz - ax, ty,
            xz - ay, yz + ax, 1.0 + f2 * (wz * wz - n), tz,
        )
        r16 = rloc * 16
        for j, val in enumerate(vals):
            plsc.store_scatter(out_v, [r16 + j], val)
        return 0

    lax.fori_loop(0, T_GROUPS, group, 0)
    pltpu.sync_copy(out_v, table_hbm.at[pl.ds(base * 16, T_PER_W * 16)])


def _gather_body(idx_hbm, table_hbm, out_hbm, idxraw_v, midx_v, rows_v, sem):
    cid = lax.axis_index("c")
    sid = lax.axis_index("s")
    wid = sid * NC + cid

    def chunk_body(ci, _):
        base = wid * IDX_PER_W + ci * CHUNK
        pltpu.sync_copy(idx_hbm.at[pl.ds(base, CHUNK)], idxraw_v)

        def map_group(g, _):
            v = idxraw_v[pl.ds(g * L, L)]
            row = jnp.where(v < BOUNDARY, v >> EXT_SHIFT,
                            v - (BOUNDARY - EXT_ROWS))
            midx_v[pl.ds(g * L, L)] = row
            return 0

        lax.fori_loop(0, CHUNK // L, map_group, 0)

        cps = [pltpu.async_copy(
            table_hbm.at[midx_v.at[pl.ds(j * GATHER, GATHER)]],
            rows_v.at[pl.ds(j * GATHER, GATHER)], sem)
            for j in range(N_GATHER)]
        for cp in cps:
            cp.wait()
        pltpu.sync_copy(rows_v, out_hbm.at[pl.ds(base, CHUNK)])
        return 0

    lax.fori_loop(0, N_CHUNKS, chunk_body, 0)


_MESH = plsc.VectorSubcoreMesh(core_axis_name="c", subcore_axis_name="s")
_PARAMS = pltpu.CompilerParams(needs_layout_passes=False,
                               use_tc_tiling_on_sc=False)

_expmap_table = functools.partial(
    pl.kernel,
    out_type=jax.ShapeDtypeStruct((T_ROWS * 16,), jnp.float32),
    mesh=_MESH,
    compiler_params=_PARAMS,
    scratch_types=[
        pltpu.VMEM((IN_ROWS * 6,), jnp.float32),
        pltpu.VMEM((T_PER_W * 16,), jnp.float32),
        pltpu.SemaphoreType.DMA,
    ],
)(_expmap_table_body)

_gather = functools.partial(
    pl.kernel,
    out_type=jax.ShapeDtypeStruct((N_IDX, 16), jnp.float32),
    mesh=_MESH,
    compiler_params=_PARAMS,
    scratch_types=[
        pltpu.VMEM((CHUNK,), jnp.int32),
        pltpu.VMEM((CHUNK,), jnp.int32),
        pltpu.VMEM((CHUNK, 16), jnp.float32),
        pltpu.SemaphoreType.DMA,
    ],
)(_gather_body)


def kernel(indices, ext_adjustment, lidar_adjustment):
    tang = jnp.concatenate(
        [ext_adjustment, lidar_adjustment,
         jnp.zeros((T_SRC - CAMERA_COUNT - SEQ_LEN, 6), ext_adjustment.dtype)],
        axis=0).reshape(T_SRC * 6)
    table = _expmap_table(tang).reshape(T_ROWS, 16)
    out = _gather(indices.astype(jnp.int32), table)
    return out[:, :12].reshape(N_IDX, 3, 4)
